# baseline (speedup 1.0000x reference)
import functools

import jax
import jax.numpy as jnp
from jax import lax
from jax.experimental import pallas as pl
from jax.experimental.pallas import tpu as pltpu
from jax.experimental.pallas import tpu_sc as plsc

N_WORKERS = 32
LANES = 16
IDX_CHUNK = 128
BLOCK_FACES = 640
ROW_W = 8


def _face_normals_sc(n_verts, n_faces):
  fpw0 = (n_faces // N_WORKERS) // 8 * 8
  fpw_last = n_faces - (N_WORKERS - 1) * fpw0
  n_blocks = -(-fpw_last // BLOCK_FACES)
  fpw_pad = n_blocks * BLOCK_FACES
  blk_idx = BLOCK_FACES * 3
  n_chunks = blk_idx // IDX_CHUNK
  n_iters = BLOCK_FACES // LANES
  tail = fpw_last - fpw0
  last_blk = fpw0 - (n_blocks - 1) * BLOCK_FACES

  mesh = plsc.VectorSubcoreMesh(core_axis_name="c", subcore_axis_name="s")

  @functools.partial(
      pl.kernel,
      mesh=mesh,
      compiler_params=pltpu.CompilerParams(
          needs_layout_passes=False, use_tc_tiling_on_sc=False),
      out_type=jax.ShapeDtypeStruct((n_faces, 3), jnp.float32),
      scratch_types=[
          pltpu.VMEM((fpw_pad, 3), jnp.int32),
          pltpu.VMEM((fpw_pad * 3,), jnp.int32),
          pltpu.VMEM((blk_idx, ROW_W), jnp.float32),
          pltpu.VMEM((blk_idx, ROW_W), jnp.float32),
          pltpu.VMEM((BLOCK_FACES, 3), jnp.float32),
          pltpu.VMEM((BLOCK_FACES, 3), jnp.float32),
          pltpu.SemaphoreType.DMA,
          pltpu.SemaphoreType.DMA,
          pltpu.SemaphoreType.DMA,
          pltpu.SemaphoreType.DMA,
      ],
  )
  def k(vert_hbm, faces_hbm, out_hbm,
        idx2d, idx_v, rows0, rows1, ob0, ob1, sem0, sem1, os0, os1):
    sid = lax.axis_index("s")
    cid = lax.axis_index("c")
    wid = sid * 2 + cid
    fbase = wid * fpw0
    shift = jnp.where(wid == N_WORKERS - 1, fpw_pad - fpw_last, 0)
    rows = (rows0, rows1)
    sems = (sem0, sem1)
    obufs = (ob0, ob1)
    osems = (os0, os1)

    pltpu.sync_copy(faces_hbm.at[pl.ds(fbase - shift, fpw_pad)], idx2d)

    iota = lax.iota(jnp.int32, LANES)
    c0 = jnp.zeros((LANES,), jnp.int32)
    c1 = c0 + 1
    c2 = c0 + 2
    max_row = jnp.full((LANES,), fpw_pad - 1, jnp.int32)

    def repack_block(b):
      def body(i, _):
        f = b * BLOCK_FACES + i * LANES + iota
        src = jnp.minimum(f + shift, max_row)
        f3 = f * 3
        i0 = plsc.load_gather(idx2d, [src, c0])
        i1 = plsc.load_gather(idx2d, [src, c1])
        i2 = plsc.load_gather(idx2d, [src, c2])
        plsc.store_scatter(idx_v, [f3], i0)
        plsc.store_scatter(idx_v, [f3 + 1], i1)
        plsc.store_scatter(idx_v, [f3 + 2], i2)
        return _
      lax.fori_loop(0, n_iters, body, None)

    def fire_block(b):
      buf, sem = rows[b % 2], sems[b % 2]
      def fire(c, _):
        src = vert_hbm.at[idx_v.at[pl.ds(b * blk_idx + c * IDX_CHUNK,
                                         IDX_CHUNK)]]
        pltpu.async_copy(src, buf.at[pl.ds(c * IDX_CHUNK, IDX_CHUNK)], sem)
        return _
      lax.fori_loop(0, n_chunks, fire, None)

    def drain_block(b):
      buf, sem = rows[b % 2], sems[b % 2]
      def drain(c, _):
        src = vert_hbm.at[idx_v.at[pl.ds(b * blk_idx + c * IDX_CHUNK,
                                         IDX_CHUNK)]]
        pltpu.make_async_copy(
            src, buf.at[pl.ds(c * IDX_CHUNK, IDX_CHUNK)], sem).wait()
        return _
      lax.fori_loop(0, n_chunks, drain, None)

    def compute_block(b):
      buf = rows[b % 2]
      obuf = obufs[b % 2]

      def body(i, _):
        lf = i * LANES + iota
        lf3 = lf * 3
        v0x = plsc.load_gather(buf, [lf3, c0])
        v0y = plsc.load_gather(buf, [lf3, c1])
        v0z = plsc.load_gather(buf, [lf3, c2])
        v1x = plsc.load_gather(buf, [lf3 + 1, c0])
        v1y = plsc.load_gather(buf, [lf3 + 1, c1])
        v1z = plsc.load_gather(buf, [lf3 + 1, c2])
        v2x = plsc.load_gather(buf, [lf3 + 2, c0])
        v2y = plsc.load_gather(buf, [lf3 + 2, c1])
        v2z = plsc.load_gather(buf, [lf3 + 2, c2])
        e1x, e1y, e1z = v1x - v0x, v1y - v0y, v1z - v0z
        e2x, e2y, e2z = v2x - v0x, v2y - v0y, v2z - v0z
        nx = e1y * e2z - e1z * e2y
        ny = e1z * e2x - e1x * e2z
        nz = e1x * e2y - e1y * e2x
        s = nx * nx + ny * ny + nz * nz
        bi = jnp.int32(0x5F3759DF) - lax.shift_right_logical(
            plsc.bitcast(s, jnp.int32), 1)
        y = plsc.bitcast(bi, jnp.float32)
        half_s = 0.5 * s
        y = y * (1.5 - half_s * y * y)
        y = y * (1.5 - half_s * y * y)
        y = y * (1.5 - half_s * y * y)
        norm = s * y
        r = 1.0 / jnp.maximum(norm, 1e-6)
        plsc.store_scatter(obuf, [lf, c0], nx * r)
        plsc.store_scatter(obuf, [lf, c1], ny * r)
        plsc.store_scatter(obuf, [lf, c2], nz * r)
        return _
      lax.fori_loop(0, n_iters, body, None)

    def out_copy(b):
      nrows = BLOCK_FACES if b < n_blocks - 1 else last_blk
      obuf = obufs[b % 2]
      return pltpu.make_async_copy(
          obuf.at[pl.ds(0, nrows)],
          out_hbm.at[pl.ds(fbase + b * BLOCK_FACES, nrows)],
          osems[b % 2])

    repack_block(0)
    fire_block(0)
    for b in range(n_blocks):
      if b + 1 < n_blocks:
        repack_block(b + 1)
        fire_block(b + 1)
      drain_block(b)
      if b >= 2:
        out_copy(b - 2).wait()
      compute_block(b)
      out_copy(b).start()
    out_copy(n_blocks - 2).wait()
    out_copy(n_blocks - 1).wait()

    @pl.when(wid == N_WORKERS - 1)
    def _():
      obuf = obufs[(n_blocks - 1) % 2]
      pltpu.sync_copy(obuf.at[pl.ds(last_blk, tail)],
                      out_hbm.at[pl.ds(fbase + fpw0, tail)])

  return k


def kernel(vertices, faces):
  n_verts = vertices.shape[1]
  n_faces = faces.shape[0]
  k = _face_normals_sc(n_verts, n_faces)

  v = jnp.pad(vertices[0], ((0, 0), (0, ROW_W - 3)))
  return k(v, faces)


# speedup vs baseline: 1.1547x; 1.1547x over previous
import functools

import jax
import jax.numpy as jnp
from jax import lax
from jax.experimental import pallas as pl
from jax.experimental.pallas import tpu as pltpu
from jax.experimental.pallas import tpu_sc as plsc

N_WORKERS = 32
LANES = 16
IDX_CHUNK = 128
BLOCK_FACES = 640
ROW_W = 8


def _face_normals_sc(n_verts, n_faces):
  fpw0 = (n_faces // N_WORKERS) // 8 * 8
  fpw_last = n_faces - (N_WORKERS - 1) * fpw0
  n_blocks = -(-fpw_last // BLOCK_FACES)
  fpw_pad = n_blocks * BLOCK_FACES
  blk_idx = BLOCK_FACES * 3
  n_chunks = blk_idx // IDX_CHUNK
  n_iters = BLOCK_FACES // LANES
  tail = fpw_last - fpw0
  last_blk = fpw0 - (n_blocks - 1) * BLOCK_FACES

  mesh = plsc.VectorSubcoreMesh(core_axis_name="c", subcore_axis_name="s")

  @functools.partial(
      pl.kernel,
      mesh=mesh,
      compiler_params=pltpu.CompilerParams(
          needs_layout_passes=False, use_tc_tiling_on_sc=False),
      out_type=jax.ShapeDtypeStruct((n_faces, 3), jnp.float32),
      scratch_types=[
          pltpu.VMEM((fpw_pad * 3 + 272,), jnp.int32),
          pltpu.VMEM((blk_idx, ROW_W), jnp.float32),
          pltpu.VMEM((blk_idx, ROW_W), jnp.float32),
          pltpu.VMEM((BLOCK_FACES, 3), jnp.float32),
          pltpu.VMEM((BLOCK_FACES, 3), jnp.float32),
          pltpu.SemaphoreType.DMA,
          pltpu.SemaphoreType.DMA,
          pltpu.SemaphoreType.DMA,
          pltpu.SemaphoreType.DMA,
      ],
  )
  def k(vert_hbm, faces_hbm, out_hbm,
        idx_v, rows0, rows1, ob0, ob1, sem0, sem1, os0, os1):
    sid = lax.axis_index("s")
    cid = lax.axis_index("c")
    wid = sid * 2 + cid
    fbase = wid * fpw0
    shift = jnp.where(wid == N_WORKERS - 1,
                      (fpw_pad - fpw_last) * 3, 0)
    rows = (rows0, rows1)
    sems = (sem0, sem1)
    obufs = (ob0, ob1)
    osems = (os0, os1)

    pltpu.sync_copy(faces_hbm.at[pl.ds(fbase * 3 - shift, fpw_pad * 3)],
                    idx_v.at[pl.ds(0, fpw_pad * 3)])

    iota = lax.iota(jnp.int32, LANES)
    c0 = jnp.zeros((LANES,), jnp.int32)
    c1 = c0 + 1
    c2 = c0 + 2

    zeros16 = jnp.zeros((LANES,), jnp.int32)
    def zfill(i, _):
      plsc.store_scatter(idx_v, [fpw_pad * 3 + i * LANES + iota], zeros16)
      return _
    lax.fori_loop(0, 272 // LANES, zfill, None)

    def fire_block(b):
      buf, sem = rows[b % 2], sems[b % 2]
      def fire(c, _):
        src = vert_hbm.at[idx_v.at[pl.ds(shift + b * blk_idx + c * IDX_CHUNK,
                                         IDX_CHUNK)]]
        pltpu.async_copy(src, buf.at[pl.ds(c * IDX_CHUNK, IDX_CHUNK)], sem)
        return _
      lax.fori_loop(0, n_chunks, fire, None)

    def drain_block(b):
      buf, sem = rows[b % 2], sems[b % 2]
      def drain(c, _):
        src = vert_hbm.at[idx_v.at[pl.ds(shift + b * blk_idx + c * IDX_CHUNK,
                                         IDX_CHUNK)]]
        pltpu.make_async_copy(
            src, buf.at[pl.ds(c * IDX_CHUNK, IDX_CHUNK)], sem).wait()
        return _
      lax.fori_loop(0, n_chunks, drain, None)

    def compute_block(b):
      buf = rows[b % 2]
      obuf = obufs[b % 2]

      def body(i, _):
        lf = i * LANES + iota
        lf3 = lf * 3
        v0x = plsc.load_gather(buf, [lf3, c0])
        v0y = plsc.load_gather(buf, [lf3, c1])
        v0z = plsc.load_gather(buf, [lf3, c2])
        v1x = plsc.load_gather(buf, [lf3 + 1, c0])
        v1y = plsc.load_gather(buf, [lf3 + 1, c1])
        v1z = plsc.load_gather(buf, [lf3 + 1, c2])
        v2x = plsc.load_gather(buf, [lf3 + 2, c0])
        v2y = plsc.load_gather(buf, [lf3 + 2, c1])
        v2z = plsc.load_gather(buf, [lf3 + 2, c2])
        e1x, e1y, e1z = v1x - v0x, v1y - v0y, v1z - v0z
        e2x, e2y, e2z = v2x - v0x, v2y - v0y, v2z - v0z
        nx = e1y * e2z - e1z * e2y
        ny = e1z * e2x - e1x * e2z
        nz = e1x * e2y - e1y * e2x
        s = nx * nx + ny * ny + nz * nz
        bi = jnp.int32(0x5F3759DF) - lax.shift_right_logical(
            plsc.bitcast(s, jnp.int32), 1)
        y = plsc.bitcast(bi, jnp.float32)
        half_s = 0.5 * s
        y = y * (1.5 - half_s * y * y)
        y = y * (1.5 - half_s * y * y)
        y = y * (1.5 - half_s * y * y)
        norm = s * y
        r = 1.0 / jnp.maximum(norm, 1e-6)
        plsc.store_scatter(obuf, [lf, c0], nx * r)
        plsc.store_scatter(obuf, [lf, c1], ny * r)
        plsc.store_scatter(obuf, [lf, c2], nz * r)
        return _
      lax.fori_loop(0, n_iters, body, None)

    def out_copy(b):
      nrows = BLOCK_FACES if b < n_blocks - 1 else last_blk
      obuf = obufs[b % 2]
      return pltpu.make_async_copy(
          obuf.at[pl.ds(0, nrows)],
          out_hbm.at[pl.ds(fbase + b * BLOCK_FACES, nrows)],
          osems[b % 2])

    fire_block(0)
    for b in range(n_blocks):
      if b + 1 < n_blocks:
        fire_block(b + 1)
      drain_block(b)
      if b >= 2:
        out_copy(b - 2).wait()
      compute_block(b)
      out_copy(b).start()
    out_copy(n_blocks - 2).wait()
    out_copy(n_blocks - 1).wait()

    @pl.when(wid == N_WORKERS - 1)
    def _():
      obuf = obufs[(n_blocks - 1) % 2]
      pltpu.sync_copy(obuf.at[pl.ds(last_blk, tail)],
                      out_hbm.at[pl.ds(fbase + fpw0, tail)])

  return k


def kernel(vertices, faces):
  n_verts = vertices.shape[1]
  n_faces = faces.shape[0]
  k = _face_normals_sc(n_verts, n_faces)

  v = jnp.pad(vertices[0], ((0, 0), (0, ROW_W - 3)))
  return k(v, faces.reshape(-1))
